# baseline (speedup 1.0000x reference)
import functools

import jax
import jax.numpy as jnp
from jax import lax
from jax.experimental import pallas as pl
from jax.experimental.pallas import tpu as pltpu
from jax.experimental.pallas import tpu_sc as plsc

VOCAB = 8192
BATCH = 4096
D = VOCAB

NUM_CORES = 2
NUM_SUBCORES = 16
NW = NUM_CORES * NUM_SUBCORES
B_PER_W = BATCH // NW
HD = D // 2
CHUNK = 8
N_CHUNKS = 2 * B_PER_W // CHUNK
NBUF = 2


def _gather_body(idx_hbm, table2_hbm, out3_hbm, idx_v, idx2_v,
                 *bufs_and_sems):
    bufs = bufs_and_sems[:NBUF]
    gsems = bufs_and_sems[NBUF:2 * NBUF]
    ssems = bufs_and_sems[2 * NBUF:3 * NBUF]

    wid = lax.axis_index("s") * NUM_CORES + lax.axis_index("c")
    base = wid * B_PER_W
    cid0 = wid * N_CHUNKS

    pltpu.sync_copy(idx_hbm.at[pl.ds(base, B_PER_W)], idx_v)

    lane = lax.iota(jnp.int32, 16)
    parity = lane & 1
    dnums = lax.GatherDimensionNumbers(
        offset_dims=(), collapsed_slice_dims=(0,), start_index_map=(0,))
    for g in range(B_PER_W // 16):
        v = idx_v[pl.ds(g * 16, 16)]
        for h in (0, 1):
            src = lax.gather(
                v, (h * 8 + (lane >> 1))[:, None],
                dimension_numbers=dnums, slice_sizes=(1,),
                mode=lax.GatherScatterMode.PROMISE_IN_BOUNDS)
            idx2_v[pl.ds(g * 32 + h * 16, 16)] = 2 * src + parity

    def start_gather(k):
        idx_sl = idx2_v.at[pl.ds(k * CHUNK, CHUNK)]
        return pltpu.async_copy(table2_hbm.at[idx_sl], bufs[k % NBUF],
                                gsems[k % NBUF])

    def start_scatter(k):
        return pltpu.async_copy(bufs[k % NBUF], out3_hbm.at[cid0 + k],
                                ssems[k % NBUF])

    gd = {}
    sd = {}
    for k in range(NBUF):
        gd[k] = start_gather(k)
    for k in range(N_CHUNKS):
        gd.pop(k).wait()
        sd[k] = start_scatter(k)
        if k + NBUF < N_CHUNKS:
            sd.pop(k).wait()
            gd[k + NBUF] = start_gather(k + NBUF)
    for k in range(N_CHUNKS - NBUF, N_CHUNKS):
        sd.pop(k).wait()


@jax.jit
def _lookup(idx, table2):
    mesh = plsc.VectorSubcoreMesh(core_axis_name="c", subcore_axis_name="s")
    kfn = functools.partial(
        pl.kernel,
        mesh=mesh,
        out_type=jax.ShapeDtypeStruct((NW * N_CHUNKS, CHUNK, HD), jnp.float32),
        scratch_types=(
            [pltpu.VMEM((B_PER_W,), jnp.int32),
             pltpu.VMEM((2 * B_PER_W,), jnp.int32)]
            + [pltpu.VMEM((CHUNK, HD), jnp.float32)] * NBUF
            + [pltpu.SemaphoreType.DMA] * (2 * NBUF)
        ),
    )(_gather_body)
    return kfn(idx, table2)


def kernel(x, table):
    last = x[:, -1].astype(jnp.int32)
    table2 = table.reshape(2 * VOCAB, HD)
    out3 = _lookup(last, table2)
    return out3.reshape(BATCH, D)


# speedup vs baseline: 4.1699x; 4.1699x over previous
import functools

import jax
import jax.numpy as jnp
from jax import lax
from jax.experimental import pallas as pl
from jax.experimental.pallas import tpu as pltpu
from jax.experimental.pallas import tpu_sc as plsc

VOCAB = 8192
BATCH = 4096
D = VOCAB

NUM_CORES = 2
NUM_SUBCORES = 16
NW = NUM_CORES * NUM_SUBCORES
B_PER_W = BATCH // NW
CHUNK = 8
N_CHUNKS = B_PER_W // CHUNK


def _gather_body(idx_hbm, table_hbm, out_hbm, idx_v, rows_v, gsem):
    wid = lax.axis_index("s") * NUM_CORES + lax.axis_index("c")
    base = wid * B_PER_W

    pltpu.sync_copy(idx_hbm.at[pl.ds(base, B_PER_W)], idx_v)

    for i in range(N_CHUNKS):
        idx_sl = idx_v.at[pl.ds(i * CHUNK, CHUNK)]
        pltpu.async_copy(table_hbm.at[idx_sl], rows_v, gsem).wait()
        pltpu.sync_copy(rows_v, out_hbm.at[pl.ds(base + i * CHUNK, CHUNK)])


@jax.jit
def _lookup(idx, table):
    mesh = plsc.VectorSubcoreMesh(core_axis_name="c", subcore_axis_name="s")
    kfn = functools.partial(
        pl.kernel,
        mesh=mesh,
        out_type=jax.ShapeDtypeStruct((BATCH, D), jnp.float32),
        scratch_types=[
            pltpu.VMEM((B_PER_W,), jnp.int32),
            pltpu.VMEM((CHUNK, D), jnp.float32),
            pltpu.SemaphoreType.DMA,
        ],
    )(_gather_body)
    return kfn(idx, table)


def kernel(x, table):
    last = x[:, -1].astype(jnp.int32)
    return _lookup(last, table)
